# baseline (speedup 1.0000x reference)
import functools

import jax
import jax.numpy as jnp
from jax import lax
from jax.experimental import pallas as pl
from jax.experimental.pallas import tpu as pltpu, tpu_sc as plsc

DIM = 64
G_MIN = 0.1

B_TOK = 4096
T_TOK = 200
C = 256
SUB = 128
NSUB = C // SUB
BBLK = B_TOK // C
NCHUNKS = T_TOK * BBLK


def _build_sc_call():
    info = plsc.get_sparse_core_info()
    nc, ns = info.num_cores, info.num_subcores
    nw = nc * ns
    per_w = NCHUNKS // nw
    npairs = per_w // 2

    mesh = plsc.VectorSubcoreMesh(core_axis_name="c", subcore_axis_name="s")

    @functools.partial(
        pl.kernel,
        mesh=mesh,
        compiler_params=pltpu.CompilerParams(
            use_tc_tiling_on_sc=False, needs_layout_passes=False),
        out_type=jax.ShapeDtypeStruct((T_TOK, DIM, B_TOK), jnp.float32),
        scratch_types=[
            pltpu.VMEM((C,), jnp.int32),
            pltpu.VMEM((C,), jnp.int32),
            pltpu.VMEM((C,), jnp.float32),
            pltpu.VMEM((C,), jnp.float32),
            pltpu.VMEM((C, DIM), jnp.float32),
            pltpu.VMEM((C, DIM), jnp.float32),
            pltpu.VMEM((C, DIM), jnp.float32),
            pltpu.VMEM((C, DIM), jnp.float32),
            pltpu.VMEM((DIM, C), jnp.float32),
            pltpu.VMEM((DIM, C), jnp.float32),
            pltpu.SemaphoreType.DMA,
            pltpu.SemaphoreType.DMA,
            pltpu.SemaphoreType.DMA,
            pltpu.SemaphoreType.DMA,
        ],
    )
    def sc_call(ids_t_h, base_h, prior_h, gate_h, out_h,
                idx0, idx1, gte0, gte1, bb0, bb1, pb0, pb1, ov0, ov1,
                gsem0, gsem1, ssem0, ssem1):
        wid = lax.axis_index("s") * nc + lax.axis_index("c")
        c_base = wid * per_w
        slots = ((idx0, gte0, bb0, pb0, ov0, gsem0, ssem0),
                 (idx1, gte1, bb1, pb1, ov1, gsem1, ssem1))

        def fire_gathers(c, slot):
            idxb, gteb, bb, pb, _, gsem, _ = slot
            t = c // BBLK
            b0 = (c % BBLK) * C
            pltpu.sync_copy(ids_t_h.at[t, pl.ds(b0, C)], idxb)
            for j in range(NSUB):
                sl = pl.ds(j * SUB, SUB)
                pltpu.async_copy(base_h.at[idxb.at[sl]], bb.at[sl, :], gsem)
                pltpu.async_copy(prior_h.at[idxb.at[sl]], pb.at[sl, :], gsem)
                pltpu.async_copy(gate_h.at[idxb.at[sl]], gteb.at[sl], gsem)

        def wait_gathers(slot):
            idxb, gteb, bb, pb, _, gsem, _ = slot
            for j in range(NSUB):
                sl = pl.ds(j * SUB, SUB)
                pltpu.make_async_copy(
                    base_h.at[idxb.at[sl]], bb.at[sl, :], gsem).wait()
                pltpu.make_async_copy(
                    prior_h.at[idxb.at[sl]], pb.at[sl, :], gsem).wait()
                pltpu.make_async_copy(
                    gate_h.at[idxb.at[sl]], gteb.at[sl], gsem).wait()

        lane = jnp.arange(16, dtype=jnp.int32)

        def compute(slot):
            _, gteb, bb, pb, ov, _, _ = slot

            def kblock(i, carry):
                k0 = i * 16
                g16 = gteb[pl.ds(k0, 16)]
                w16 = G_MIN + (1.0 - G_MIN) / (1.0 + jnp.exp(-g16))
                rows = lane + k0
                for d in range(DIM):
                    cols = jnp.full((16,), d, jnp.int32)
                    b16 = plsc.load_gather(bb, [rows, cols])
                    p16 = plsc.load_gather(pb, [rows, cols])
                    ov[d, pl.ds(k0, 16)] = b16 + w16 * p16
                return carry

            lax.fori_loop(0, C // 16, kblock, 0)

        def fire_store(c, slot):
            _, _, _, _, ov, _, ssem = slot
            t = c // BBLK
            b0 = (c % BBLK) * C
            pltpu.async_copy(ov, out_h.at[t, :, pl.ds(b0, C)], ssem)

        def wait_store(c, slot):
            _, _, _, _, ov, _, ssem = slot
            t = c // BBLK
            b0 = (c % BBLK) * C
            pltpu.make_async_copy(ov, out_h.at[t, :, pl.ds(b0, C)], ssem).wait()

        fire_gathers(c_base + 0, slots[0])
        fire_gathers(c_base + 1, slots[1])

        def pair(p, carry):
            c0 = c_base + 2 * p
            c1 = c0 + 1
            wait_gathers(slots[0])
            compute(slots[0])
            fire_store(c0, slots[0])

            @pl.when(p < npairs - 1)
            def _():
                wait_store(c0, slots[0])
                fire_gathers(c0 + 2, slots[0])

            wait_gathers(slots[1])
            compute(slots[1])
            fire_store(c1, slots[1])

            @pl.when(p < npairs - 1)
            def _():
                wait_store(c1, slots[1])
                fire_gathers(c1 + 2, slots[1])

            return carry

        lax.fori_loop(0, npairs, pair, 0)
        wait_store(c_base + per_w - 2, slots[0])
        wait_store(c_base + per_w - 1, slots[1])

    return sc_call


_SC_CALL = _build_sc_call()


@jax.jit
def kernel(input_ids, base_weight, prior_matrix, gate_logits):
    ids_t = input_ids.T.astype(jnp.int32)
    out_t = _SC_CALL(ids_t, base_weight, prior_matrix, gate_logits)
    return jnp.transpose(out_t, (2, 0, 1))


# speedup vs baseline: 1.8132x; 1.8132x over previous
import functools

import jax
import jax.numpy as jnp
from jax import lax
from jax.experimental import pallas as pl
from jax.experimental.pallas import tpu as pltpu, tpu_sc as plsc

DIM = 64
G_MIN = 0.1

B_TOK = 4096
T_TOK = 200
C = 256
SUB = 128
NSUB = C // SUB
BBLK = B_TOK // C
NCHUNKS = T_TOK * BBLK


def _build_sc_call():
    info = plsc.get_sparse_core_info()
    nc, ns = info.num_cores, info.num_subcores
    nw = nc * ns
    per_w = NCHUNKS // nw
    npairs = per_w // 2

    mesh = plsc.VectorSubcoreMesh(core_axis_name="c", subcore_axis_name="s")

    @functools.partial(
        pl.kernel,
        mesh=mesh,
        compiler_params=pltpu.CompilerParams(
            use_tc_tiling_on_sc=False, needs_layout_passes=False),
        out_type=jax.ShapeDtypeStruct((T_TOK, DIM, B_TOK), jnp.float32),
        scratch_types=[
            pltpu.VMEM((C,), jnp.int32),
            pltpu.VMEM((C,), jnp.int32),
            pltpu.VMEM((C,), jnp.float32),
            pltpu.VMEM((C,), jnp.float32),
            pltpu.VMEM((C, DIM), jnp.float32),
            pltpu.VMEM((C, DIM), jnp.float32),
            pltpu.VMEM((C, DIM), jnp.float32),
            pltpu.VMEM((C, DIM), jnp.float32),
            pltpu.VMEM((DIM, C + 1), jnp.float32),
            pltpu.VMEM((DIM, C + 1), jnp.float32),
            pltpu.SemaphoreType.DMA,
            pltpu.SemaphoreType.DMA,
            pltpu.SemaphoreType.DMA,
            pltpu.SemaphoreType.DMA,
        ],
    )
    def sc_call(ids_t_h, base_h, prior_h, gate_h, out_h,
                idx0, idx1, gte0, gte1, bb0, bb1, pb0, pb1, ov0, ov1,
                gsem0, gsem1, ssem0, ssem1):
        wid = lax.axis_index("s") * nc + lax.axis_index("c")
        c_base = wid * per_w
        slots = ((idx0, gte0, bb0, pb0, ov0, gsem0, ssem0),
                 (idx1, gte1, bb1, pb1, ov1, gsem1, ssem1))

        def fire_gathers(c, slot):
            idxb, gteb, bb, pb, _, gsem, _ = slot
            t = c // BBLK
            b0 = (c % BBLK) * C
            pltpu.sync_copy(ids_t_h.at[t, pl.ds(b0, C)], idxb)
            for j in range(NSUB):
                sl = pl.ds(j * SUB, SUB)
                pltpu.async_copy(base_h.at[idxb.at[sl]], bb.at[sl, :], gsem)
                pltpu.async_copy(prior_h.at[idxb.at[sl]], pb.at[sl, :], gsem)
                pltpu.async_copy(gate_h.at[idxb.at[sl]], gteb.at[sl], gsem)

        def wait_gathers(slot):
            idxb, gteb, bb, pb, _, gsem, _ = slot
            for j in range(NSUB):
                sl = pl.ds(j * SUB, SUB)
                pltpu.make_async_copy(
                    base_h.at[idxb.at[sl]], bb.at[sl, :], gsem).wait()
                pltpu.make_async_copy(
                    prior_h.at[idxb.at[sl]], pb.at[sl, :], gsem).wait()
                pltpu.make_async_copy(
                    gate_h.at[idxb.at[sl]], gteb.at[sl], gsem).wait()

        lane = jnp.arange(16, dtype=jnp.int32)
        dnums = lax.GatherDimensionNumbers(
            offset_dims=(), collapsed_slice_dims=(0,), start_index_map=(0,))

        def compute(slot):
            _, gteb, bb, pb, ov, _, _ = slot

            def kblock(i, carry):
                k0 = i * 16
                g16 = gteb[pl.ds(k0, 16)]
                w16 = G_MIN + (1.0 - G_MIN) / (1.0 + jnp.exp(-g16))
                for r in range(16):
                    wr = lax.gather(
                        w16, jnp.full((16, 1), r, jnp.int32), dnums, (1,),
                        mode=lax.GatherScatterMode.PROMISE_IN_BOUNDS)
                    row = k0 + r
                    col = jnp.full((16,), 0, jnp.int32) + row
                    for dc in range(DIM // 16):
                        dsl = pl.ds(dc * 16, 16)
                        v = bb[row, dsl] + wr * pb[row, dsl]
                        plsc.store_scatter(ov, [lane + dc * 16, col], v)
                return carry

            lax.fori_loop(0, C // 16, kblock, 0)

        def fire_store(c, slot):
            _, _, _, _, ov, _, ssem = slot
            t = c // BBLK
            b0 = (c % BBLK) * C
            pltpu.async_copy(
                ov.at[:, pl.ds(0, C)], out_h.at[t, :, pl.ds(b0, C)], ssem)

        def wait_store(c, slot):
            _, _, _, _, ov, _, ssem = slot
            t = c // BBLK
            b0 = (c % BBLK) * C
            pltpu.make_async_copy(
                ov.at[:, pl.ds(0, C)], out_h.at[t, :, pl.ds(b0, C)], ssem).wait()

        fire_gathers(c_base + 0, slots[0])
        fire_gathers(c_base + 1, slots[1])

        def pair(p, carry):
            c0 = c_base + 2 * p
            c1 = c0 + 1
            wait_gathers(slots[0])
            compute(slots[0])
            fire_store(c0, slots[0])

            @pl.when(p < npairs - 1)
            def _():
                wait_store(c0, slots[0])
                fire_gathers(c0 + 2, slots[0])

            wait_gathers(slots[1])
            compute(slots[1])
            fire_store(c1, slots[1])

            @pl.when(p < npairs - 1)
            def _():
                wait_store(c1, slots[1])
                fire_gathers(c1 + 2, slots[1])

            return carry

        lax.fori_loop(0, npairs, pair, 0)
        wait_store(c_base + per_w - 2, slots[0])
        wait_store(c_base + per_w - 1, slots[1])

    return sc_call


_SC_CALL = _build_sc_call()


@jax.jit
def kernel(input_ids, base_weight, prior_matrix, gate_logits):
    ids_t = input_ids.T.astype(jnp.int32)
    out_t = _SC_CALL(ids_t, base_weight, prior_matrix, gate_logits)
    return jnp.transpose(out_t, (2, 0, 1))
